# baseline (speedup 1.0000x reference)
import functools

import jax
import jax.numpy as jnp
from jax import lax
from jax.experimental import pallas as pl
from jax.experimental.pallas import tpu as pltpu
from jax.experimental.pallas import tpu_sc as plsc

N = 10000
E = 320000
NC = 2
NS = 16
NW = NC * NS
C = 80
EPS = E // NS
EPSP = EPS
NCHS = EPSP // C
PAD_DST = 10016
NPAD = 10240
NPT = NPAD // NS
ZR = 128


def _zero_fill(zbuf, acc, s, d):
    def zrow(r, _):
        for kk in range(d // 16):
            zbuf[r, pl.ds(kk * 16, 16)] = jnp.zeros((16,), jnp.float32)
        return 0

    lax.fori_loop(0, ZR, zrow, 0)

    def zcp(t, _):
        pltpu.sync_copy(zbuf, acc.at[pl.ds(s * NPT + t * ZR, ZR)])
        return 0

    lax.fori_loop(0, NPT // ZR, zcp, 0)


def _pipeline(run_chunks, hs_hbm, src_v, dst_v, b0, b1, acc, g0, g1):
    def fire(j, buf, sem):
        pltpu.async_copy(hs_hbm.at[src_v.at[j]], buf, sem)

    def drain(j, buf, sem):
        pltpu.make_async_copy(hs_hbm.at[src_v.at[j]], buf, sem).wait()
        pltpu.sync_copy(buf, acc.at[dst_v.at[j]], add=True)

    fire(0, b0, g0)
    fire(1, b1, g1)

    def pair(i, _):
        drain(2 * i, b0, g0)
        fire(2 * i + 2, b0, g0)
        drain(2 * i + 1, b1, g1)
        fire(2 * i + 3, b1, g1)
        return 0

    lax.fori_loop(0, run_chunks // 2 - 1, pair, 0)
    drain(run_chunks - 2, b0, g0)
    drain(run_chunks - 1, b1, g1)


def _make_deg():
    mesh = plsc.VectorSubcoreMesh(core_axis_name="c", subcore_axis_name="s")

    @functools.partial(
        pl.kernel,
        out_type=jax.ShapeDtypeStruct((NC * NPAD, 16), jnp.float32),
        mesh=mesh,
        scratch_types=[
            pltpu.VMEM((NCHS, C), jnp.int32),
            pltpu.VMEM((C, 16), jnp.float32),
            pltpu.VMEM((ZR, 16), jnp.float32),
            pltpu.VMEM_SHARED((NPAD, 16), jnp.float32),
        ],
        compiler_params=pltpu.CompilerParams(use_tc_tiling_on_sc=False),
    )
    def deg(eidx_hbm, out_hbm, dst_v, ones_v, zbuf, acc):
        c = lax.axis_index("c")
        s = lax.axis_index("s")
        pltpu.sync_copy(eidx_hbm.at[1, s], dst_v)

        def orow(r, _):
            ones_v[r, :] = jnp.ones((16,), jnp.float32)
            return 0

        lax.fori_loop(0, C, orow, 0)
        _zero_fill(zbuf, acc, s, 16)
        plsc.subcore_barrier()

        base = c * (NCHS // 2)

        def chunk(j, _):
            pltpu.sync_copy(ones_v, acc.at[dst_v.at[base + j]], add=True)
            return 0

        lax.fori_loop(0, NCHS // 2, chunk, 0)
        plsc.subcore_barrier()
        pltpu.sync_copy(
            acc.at[pl.ds(s * NPT, NPT)],
            out_hbm.at[pl.ds(c * NPAD + s * NPT, NPT)],
        )

    return deg


def _make_agg_cols(dh):
    mesh = plsc.VectorSubcoreMesh(core_axis_name="c", subcore_axis_name="s")

    @functools.partial(
        pl.kernel,
        out_type=jax.ShapeDtypeStruct((NC * NPAD, dh), jnp.float32),
        mesh=mesh,
        scratch_types=[
            pltpu.VMEM((NCHS, C), jnp.int32),
            pltpu.VMEM((NCHS, C), jnp.int32),
            pltpu.VMEM((C, dh), jnp.float32),
            pltpu.VMEM((C, dh), jnp.float32),
            pltpu.VMEM((ZR, dh), jnp.float32),
            pltpu.VMEM_SHARED((NPAD, dh), jnp.float32),
            pltpu.SemaphoreType.DMA,
            pltpu.SemaphoreType.DMA,
        ],
        compiler_params=pltpu.CompilerParams(use_tc_tiling_on_sc=False),
    )
    def agg(hsa_hbm, hsb_hbm, eidx_hbm, out_hbm,
            src_v, dst_v, b0, b1, zbuf, acc, g0, g1):
        c = lax.axis_index("c")
        s = lax.axis_index("s")
        pltpu.sync_copy(eidx_hbm.at[0, s], src_v)
        pltpu.sync_copy(eidx_hbm.at[1, s], dst_v)
        _zero_fill(zbuf, acc, s, dh)
        plsc.subcore_barrier()

        @pl.when(c == 0)
        def _():
            _pipeline(NCHS, hsa_hbm, src_v, dst_v, b0, b1, acc, g0, g1)

        @pl.when(c == 1)
        def _():
            _pipeline(NCHS, hsb_hbm, src_v, dst_v, b0, b1, acc, g0, g1)

        plsc.subcore_barrier()
        pltpu.sync_copy(
            acc.at[pl.ds(s * NPT, NPT)],
            out_hbm.at[pl.ds(c * NPAD + s * NPT, NPT)],
        )

    return agg


_deg_kernel = _make_deg()
_agg_l1 = _make_agg_cols(32)
_agg_l2 = _make_agg_cols(64)


_R = 1280
_G = NPAD // _R


def _tc1_body(c0, c1, x_ref, w1, dinv_ref, hsa_ref, hsb_ref):
    deg = c0[:, 0:1] + c1[:, 0:1] + 1.0
    dinv = lax.rsqrt(deg)
    h = jnp.dot(x_ref[...], w1[...], preferred_element_type=jnp.float32)
    hs = h * dinv
    dinv_ref[...] = dinv
    hsa_ref[...] = hs[:, :32]
    hsb_ref[...] = hs[:, 32:]


def _tc1(counts, x, w1):
    return pl.pallas_call(
        _tc1_body,
        grid=(_G,),
        in_specs=[
            pl.BlockSpec((_R, 16), lambda i: (i, 0)),
            pl.BlockSpec((_R, 16), lambda i: (i + _G, 0)),
            pl.BlockSpec((_R, 128), lambda i: (i, 0)),
            pl.BlockSpec((128, 64), lambda i: (0, 0)),
        ],
        out_specs=[
            pl.BlockSpec((_R, 1), lambda i: (i, 0)),
            pl.BlockSpec((_R, 32), lambda i: (i, 0)),
            pl.BlockSpec((_R, 32), lambda i: (i, 0)),
        ],
        out_shape=[
            jax.ShapeDtypeStruct((NPAD, 1), jnp.float32),
            jax.ShapeDtypeStruct((NPAD, 32), jnp.float32),
            jax.ShapeDtypeStruct((NPAD, 32), jnp.float32),
        ],
    )(counts, counts, x, w1)


def _tc2_body(ua, ub, hsa, hsb, dinv_ref, b1, w2, hs2a_ref, hs2b_ref):
    dinv = dinv_ref[...]
    u = jnp.concatenate([ua[...], ub[...]], axis=1)
    hs = jnp.concatenate([hsa[...], hsb[...]], axis=1)
    a = dinv * (u + hs) + b1[...]
    r = jnp.maximum(a, 0.0)
    h2 = jnp.dot(r, w2[...], preferred_element_type=jnp.float32)
    hs2 = h2 * dinv
    hs2a_ref[...] = hs2[:, :64]
    hs2b_ref[...] = hs2[:, 64:]


def _tc2(u, hsa, hsb, dinv, b1, w2):
    return pl.pallas_call(
        _tc2_body,
        grid=(_G,),
        in_specs=[
            pl.BlockSpec((_R, 32), lambda i: (i, 0)),
            pl.BlockSpec((_R, 32), lambda i: (i + _G, 0)),
            pl.BlockSpec((_R, 32), lambda i: (i, 0)),
            pl.BlockSpec((_R, 32), lambda i: (i, 0)),
            pl.BlockSpec((_R, 1), lambda i: (i, 0)),
            pl.BlockSpec((1, 64), lambda i: (0, 0)),
            pl.BlockSpec((64, 128), lambda i: (0, 0)),
        ],
        out_specs=[
            pl.BlockSpec((_R, 64), lambda i: (i, 0)),
            pl.BlockSpec((_R, 64), lambda i: (i, 0)),
        ],
        out_shape=[
            jax.ShapeDtypeStruct((NPAD, 64), jnp.float32),
            jax.ShapeDtypeStruct((NPAD, 64), jnp.float32),
        ],
    )(u, u, hsa, hsb, dinv, b1, w2)


def _tc3_body(ua, ub, hsa, hsb, dinv_ref, b2, wfc, bfc, out_ref):
    dinv = dinv_ref[...]
    u = jnp.concatenate([ua[...], ub[...]], axis=1)
    hs = jnp.concatenate([hsa[...], hsb[...]], axis=1)
    a = dinv * (u + hs) + b2[...]
    r = jnp.maximum(a, 0.0)
    out_ref[...] = jnp.dot(r, wfc[...], preferred_element_type=jnp.float32) + bfc[...]


def _tc3(u, hsa, hsb, dinv, b2, wfc, bfc):
    return pl.pallas_call(
        _tc3_body,
        grid=(_G,),
        in_specs=[
            pl.BlockSpec((_R, 64), lambda i: (i, 0)),
            pl.BlockSpec((_R, 64), lambda i: (i + _G, 0)),
            pl.BlockSpec((_R, 64), lambda i: (i, 0)),
            pl.BlockSpec((_R, 64), lambda i: (i, 0)),
            pl.BlockSpec((_R, 1), lambda i: (i, 0)),
            pl.BlockSpec((1, 128), lambda i: (0, 0)),
            pl.BlockSpec((128, 1), lambda i: (0, 0)),
            pl.BlockSpec((1, 1), lambda i: (0, 0)),
        ],
        out_specs=pl.BlockSpec((_R, 1), lambda i: (i, 0)),
        out_shape=jax.ShapeDtypeStruct((NPAD, 1), jnp.float32),
    )(u, u, hsa, hsb, dinv, b2, wfc, bfc)


def kernel(x, edge_index, W1, b1, W2, b2, Wfc, bfc):
    ei = edge_index.astype(jnp.int32).reshape(2, NS, EPS)
    if EPSP == EPS:
        eidx = ei.reshape(2, NS, NCHS, C)
    else:
        srcp = jnp.pad(ei[0], ((0, 0), (0, EPSP - EPS)))
        dstp = jnp.pad(ei[1], ((0, 0), (0, EPSP - EPS)),
                       constant_values=PAD_DST)
        eidx = jnp.stack([srcp, dstp]).reshape(2, NS, NCHS, C)

    counts = _deg_kernel(eidx)
    dinv, hsa, hsb = _tc1(counts, x, W1)
    u1 = _agg_l1(hsa, hsb, eidx)
    hs2a, hs2b = _tc2(u1, hsa, hsb, dinv, b1.reshape(1, 64), W2)
    u2 = _agg_l2(hs2a, hs2b, eidx)
    outp = _tc3(u2, hs2a, hs2b, dinv, b2.reshape(1, 128), Wfc, bfc.reshape(1, 1))
    return outp[:N]


# speedup vs baseline: 1.0159x; 1.0159x over previous
import functools

import jax
import jax.numpy as jnp
from jax import lax
from jax.experimental import pallas as pl
from jax.experimental.pallas import tpu as pltpu
from jax.experimental.pallas import tpu_sc as plsc

N = 10000
E = 320000
NC = 2
NS = 16
NW = NC * NS
C = 80
EPS = E // NS
EPSP = EPS
NCHS = EPSP // C
PAD_DST = 10016
NPAD = 10240
NPT = NPAD // NS
ZR = 128


def _zero_fill(zbuf, acc, s, d):
    def zrow(r, _):
        for kk in range(d // 16):
            zbuf[r, pl.ds(kk * 16, 16)] = jnp.zeros((16,), jnp.float32)
        return 0

    lax.fori_loop(0, ZR, zrow, 0)

    def zcp(t, _):
        pltpu.sync_copy(zbuf, acc.at[pl.ds(s * NPT + t * ZR, ZR)])
        return 0

    lax.fori_loop(0, NPT // ZR, zcp, 0)


def _pipeline(run_chunks, hs_hbm, src_v, dst_v, b0, b1, acc, g0, g1):
    def fire(j, buf, sem):
        pltpu.async_copy(hs_hbm.at[src_v.at[j]], buf, sem)

    def drain(j, buf, sem):
        pltpu.make_async_copy(hs_hbm.at[src_v.at[j]], buf, sem).wait()
        pltpu.sync_copy(buf, acc.at[dst_v.at[j]], add=True)

    fire(0, b0, g0)
    fire(1, b1, g1)

    def pair(i, _):
        drain(2 * i, b0, g0)
        fire(2 * i + 2, b0, g0)
        drain(2 * i + 1, b1, g1)
        fire(2 * i + 3, b1, g1)
        return 0

    lax.fori_loop(0, run_chunks // 2 - 1, pair, 0)
    drain(run_chunks - 2, b0, g0)
    drain(run_chunks - 1, b1, g1)


def _make_deg():
    mesh = plsc.VectorSubcoreMesh(core_axis_name="c", subcore_axis_name="s")

    @functools.partial(
        pl.kernel,
        out_type=jax.ShapeDtypeStruct((NC * NPAD, 16), jnp.float32),
        mesh=mesh,
        scratch_types=[
            pltpu.VMEM((NCHS, C), jnp.int32),
            pltpu.VMEM((C, 16), jnp.float32),
            pltpu.VMEM((ZR, 16), jnp.float32),
            pltpu.VMEM_SHARED((NPAD, 16), jnp.float32),
        ],
        compiler_params=pltpu.CompilerParams(use_tc_tiling_on_sc=False),
    )
    def deg(eidx_hbm, out_hbm, dst_v, ones_v, zbuf, acc):
        c = lax.axis_index("c")
        s = lax.axis_index("s")
        pltpu.sync_copy(eidx_hbm.at[1, s], dst_v)

        def orow(r, _):
            ones_v[r, :] = jnp.ones((16,), jnp.float32)
            return 0

        lax.fori_loop(0, C, orow, 0)
        _zero_fill(zbuf, acc, s, 16)
        plsc.subcore_barrier()

        base = c * (NCHS // 2)

        def chunk(j, _):
            pltpu.sync_copy(ones_v, acc.at[dst_v.at[base + j]], add=True)
            return 0

        lax.fori_loop(0, NCHS // 2, chunk, 0)
        plsc.subcore_barrier()
        pltpu.sync_copy(
            acc.at[pl.ds(s * NPT, NPT)],
            out_hbm.at[pl.ds(c * NPAD + s * NPT, NPT)],
        )

    return deg


def _make_agg_cols(dh):
    mesh = plsc.VectorSubcoreMesh(core_axis_name="c", subcore_axis_name="s")

    @functools.partial(
        pl.kernel,
        out_type=jax.ShapeDtypeStruct((NC * NPAD, dh), jnp.float32),
        mesh=mesh,
        scratch_types=[
            pltpu.VMEM((NCHS, C), jnp.int32),
            pltpu.VMEM((NCHS, C), jnp.int32),
            pltpu.VMEM((C, dh), jnp.float32),
            pltpu.VMEM((C, dh), jnp.float32),
            pltpu.VMEM((ZR, dh), jnp.float32),
            pltpu.VMEM_SHARED((NPAD, dh), jnp.float32),
            pltpu.SemaphoreType.DMA,
            pltpu.SemaphoreType.DMA,
        ],
        compiler_params=pltpu.CompilerParams(use_tc_tiling_on_sc=False),
    )
    def agg(hsa_hbm, hsb_hbm, eidx_hbm, out_hbm,
            src_v, dst_v, b0, b1, zbuf, acc, g0, g1):
        c = lax.axis_index("c")
        s = lax.axis_index("s")
        pltpu.sync_copy(eidx_hbm.at[0, s], src_v)
        pltpu.sync_copy(eidx_hbm.at[1, s], dst_v)
        _zero_fill(zbuf, acc, s, dh)
        plsc.subcore_barrier()

        @pl.when(c == 0)
        def _():
            _pipeline(NCHS, hsa_hbm, src_v, dst_v, b0, b1, acc, g0, g1)

        @pl.when(c == 1)
        def _():
            _pipeline(NCHS, hsb_hbm, src_v, dst_v, b0, b1, acc, g0, g1)

        plsc.subcore_barrier()
        pltpu.sync_copy(
            acc.at[pl.ds(s * NPT, NPT)],
            out_hbm.at[pl.ds(c * NPAD + s * NPT, NPT)],
        )

    return agg


_deg_kernel = _make_deg()
_agg_l1 = _make_agg_cols(32)
_agg_l2 = _make_agg_cols(64)


_R = 5120
_G = NPAD // _R


def _tc1_body(c0, c1, x_ref, w1, dinv_ref, hsa_ref, hsb_ref):
    deg = c0[:, 0:1] + c1[:, 0:1] + 1.0
    dinv = lax.rsqrt(deg)
    h = jnp.dot(x_ref[...], w1[...], preferred_element_type=jnp.float32)
    hs = h * dinv
    dinv_ref[...] = dinv
    hsa_ref[...] = hs[:, :32]
    hsb_ref[...] = hs[:, 32:]


def _tc1(counts, x, w1):
    return pl.pallas_call(
        _tc1_body,
        grid=(_G,),
        in_specs=[
            pl.BlockSpec((_R, 16), lambda i: (i, 0)),
            pl.BlockSpec((_R, 16), lambda i: (i + _G, 0)),
            pl.BlockSpec((_R, 128), lambda i: (i, 0)),
            pl.BlockSpec((128, 64), lambda i: (0, 0)),
        ],
        out_specs=[
            pl.BlockSpec((_R, 1), lambda i: (i, 0)),
            pl.BlockSpec((_R, 32), lambda i: (i, 0)),
            pl.BlockSpec((_R, 32), lambda i: (i, 0)),
        ],
        out_shape=[
            jax.ShapeDtypeStruct((NPAD, 1), jnp.float32),
            jax.ShapeDtypeStruct((NPAD, 32), jnp.float32),
            jax.ShapeDtypeStruct((NPAD, 32), jnp.float32),
        ],
    )(counts, counts, x, w1)


def _tc2_body(ua, ub, hsa, hsb, dinv_ref, b1, w2, hs2a_ref, hs2b_ref):
    dinv = dinv_ref[...]
    u = jnp.concatenate([ua[...], ub[...]], axis=1)
    hs = jnp.concatenate([hsa[...], hsb[...]], axis=1)
    a = dinv * (u + hs) + b1[...]
    r = jnp.maximum(a, 0.0)
    h2 = jnp.dot(r, w2[...], preferred_element_type=jnp.float32)
    hs2 = h2 * dinv
    hs2a_ref[...] = hs2[:, :64]
    hs2b_ref[...] = hs2[:, 64:]


def _tc2(u, hsa, hsb, dinv, b1, w2):
    return pl.pallas_call(
        _tc2_body,
        grid=(_G,),
        in_specs=[
            pl.BlockSpec((_R, 32), lambda i: (i, 0)),
            pl.BlockSpec((_R, 32), lambda i: (i + _G, 0)),
            pl.BlockSpec((_R, 32), lambda i: (i, 0)),
            pl.BlockSpec((_R, 32), lambda i: (i, 0)),
            pl.BlockSpec((_R, 1), lambda i: (i, 0)),
            pl.BlockSpec((1, 64), lambda i: (0, 0)),
            pl.BlockSpec((64, 128), lambda i: (0, 0)),
        ],
        out_specs=[
            pl.BlockSpec((_R, 64), lambda i: (i, 0)),
            pl.BlockSpec((_R, 64), lambda i: (i, 0)),
        ],
        out_shape=[
            jax.ShapeDtypeStruct((NPAD, 64), jnp.float32),
            jax.ShapeDtypeStruct((NPAD, 64), jnp.float32),
        ],
    )(u, u, hsa, hsb, dinv, b1, w2)


def _tc3_body(ua, ub, hsa, hsb, dinv_ref, b2, wfc, bfc, out_ref):
    dinv = dinv_ref[...]
    u = jnp.concatenate([ua[...], ub[...]], axis=1)
    hs = jnp.concatenate([hsa[...], hsb[...]], axis=1)
    a = dinv * (u + hs) + b2[...]
    r = jnp.maximum(a, 0.0)
    out_ref[...] = jnp.dot(r, wfc[...], preferred_element_type=jnp.float32) + bfc[...]


def _tc3(u, hsa, hsb, dinv, b2, wfc, bfc):
    return pl.pallas_call(
        _tc3_body,
        grid=(_G,),
        in_specs=[
            pl.BlockSpec((_R, 64), lambda i: (i, 0)),
            pl.BlockSpec((_R, 64), lambda i: (i + _G, 0)),
            pl.BlockSpec((_R, 64), lambda i: (i, 0)),
            pl.BlockSpec((_R, 64), lambda i: (i, 0)),
            pl.BlockSpec((_R, 1), lambda i: (i, 0)),
            pl.BlockSpec((1, 128), lambda i: (0, 0)),
            pl.BlockSpec((128, 1), lambda i: (0, 0)),
            pl.BlockSpec((1, 1), lambda i: (0, 0)),
        ],
        out_specs=pl.BlockSpec((_R, 1), lambda i: (i, 0)),
        out_shape=jax.ShapeDtypeStruct((NPAD, 1), jnp.float32),
    )(u, u, hsa, hsb, dinv, b2, wfc, bfc)


def kernel(x, edge_index, W1, b1, W2, b2, Wfc, bfc):
    ei = edge_index.astype(jnp.int32).reshape(2, NS, EPS)
    if EPSP == EPS:
        eidx = ei.reshape(2, NS, NCHS, C)
    else:
        srcp = jnp.pad(ei[0], ((0, 0), (0, EPSP - EPS)))
        dstp = jnp.pad(ei[1], ((0, 0), (0, EPSP - EPS)),
                       constant_values=PAD_DST)
        eidx = jnp.stack([srcp, dstp]).reshape(2, NS, NCHS, C)

    counts = _deg_kernel(eidx)
    dinv, hsa, hsb = _tc1(counts, x, W1)
    u1 = _agg_l1(hsa, hsb, eidx)
    hs2a, hs2b = _tc2(u1, hsa, hsb, dinv, b1.reshape(1, 64), W2)
    u2 = _agg_l2(hs2a, hs2b, eidx)
    outp = _tc3(u2, hs2a, hs2b, dinv, b2.reshape(1, 128), Wfc, bfc.reshape(1, 1))
    return outp[:N]
